# baseline (speedup 1.0000x reference)
import numpy as np
import jax
import jax.numpy as jnp
from jax.experimental import pallas as pl
from jax.experimental.pallas import tpu as pltpu

_NUM_LEVELS = 16
_BASE_RES = 16
_DESIRED_RES = 4096
_LOG2_T = 19
_T = 2 ** _LOG2_T
_SCALE = float(np.exp(np.log(_DESIRED_RES / _BASE_RES) / (_NUM_LEVELS - 1)))
_RES = [int(np.ceil(_BASE_RES * _SCALE ** l)) for l in range(_NUM_LEVELS)]
_SIZES = [min(r ** 3, _T) for r in _RES]
_ROWS = [(2 * s + 127) // 128 for s in _SIZES]
_P1 = np.uint32(2654435761)
_P2 = np.uint32(805459861)


def _encode_mlp_kernel(x_ref, y_ref, z_ref, nrm_ref, mmn_ref, mms_ref,
                       g1_ref, w2_ref, w3_ref, *rest):
    tabs = rest[:_NUM_LEVELS]
    out_ref = rest[_NUM_LEVELS]
    (pk_ref, acc_ref, pk_smem, sem1) = rest[_NUM_LEVELS + 1:]

    b8 = x_ref.shape[1]
    npts = b8 * 128

    coords = []
    for d, ref in enumerate((x_ref, y_ref, z_ref)):
        c = ref[0]
        c = jnp.clip((c - nrm_ref[0, d]) * nrm_ref[1, d], 0.0, 1.0)
        coords.append(c)

    for l in range(_NUM_LEVELS):
        res = _RES[l]
        dense = res ** 3 <= _T
        pgs, frs = [], []
        for c in coords:
            pos = c * (res - 1)
            pg = jnp.floor(pos)
            frs.append(pos - pg)
            pgs.append(pg.astype(jnp.int32))
        omf = [1.0 - f for f in frs]
        for corner in range(8):
            offs = [(corner >> d) & 1 for d in range(3)]
            cc = [jnp.clip(pgs[d] + offs[d], 0, res - 1) for d in range(3)]
            if dense:
                e = cc[0] + res * (cc[1] + res * cc[2])
            else:
                h = (cc[0].astype(jnp.uint32)
                     ^ (cc[1].astype(jnp.uint32) * _P1)
                     ^ (cc[2].astype(jnp.uint32) * _P2))
                e = (h & np.uint32(_T - 1)).astype(jnp.int32)
            w = ((frs[0] if offs[0] else omf[0])
                 * (frs[1] if offs[1] else omf[1])
                 * (frs[2] if offs[2] else omf[2]))
            wq = jnp.round(w * 4095.0).astype(jnp.int32)
            pk_ref[l, corner] = ((e & 63) << 26) | (wq << 14) | (e >> 6)

    cp1 = pltpu.make_async_copy(pk_ref, pk_smem, sem1)
    cp1.start()
    cp1.wait()

    iotah = jax.lax.broadcasted_iota(jnp.int32, (1, 128), 1) >> 1

    def point_body(p, _):
        s = p >> 7
        li = p & 127
        for l in range(_NUM_LEVELS):
            cs = []
            for corner in range(8):
                pk = pk_smem[l, corner, s, li]
                row = tabs[l][pk & 16383]
                sel = iotah == jax.lax.shift_right_logical(pk, 26)
                wq = (pk >> 14) & 4095
                cs.append(jnp.where(sel, wq, 0).astype(jnp.float32) * row)
            acc = ((cs[0] + cs[1]) + (cs[2] + cs[3])) \
                + ((cs[4] + cs[5]) + (cs[6] + cs[7]))
            acc_ref[pl.ds(l * npts + p, 1), :] = acc * np.float32(1.0 / 4095.0)
        return _

    jax.lax.fori_loop(0, npts, point_body, None)

    h = jnp.zeros((npts, 32), jnp.float32)
    for l in range(_NUM_LEVELS):
        h = h + jnp.dot(acc_ref[l * npts:(l + 1) * npts, :],
                        g1_ref[l * 128:(l + 1) * 128, :],
                        preferred_element_type=jnp.float32)
    h = jnp.maximum(h, 0.0)
    h = jnp.maximum(jnp.dot(h, w2_ref[:], preferred_element_type=jnp.float32), 0.0)
    o = jnp.dot(h, w3_ref[:], preferred_element_type=jnp.float32)
    o = jax.nn.sigmoid(o)
    out_ref[:] = o * mms_ref[:] + mmn_ref[:]


def kernel(texc, aabb_min, aabb_max, embeddings, W1, W2, W3, mm_min, mm_max):
    lead_shape = texc.shape[:-1]
    t = texc.reshape(-1, 3)
    n = t.shape[0]
    b = 512 if n % 512 == 0 else 128
    b8 = b // 128
    nb = n // b

    xs = t[:, 0].reshape(nb, b8, 128)
    ys = t[:, 1].reshape(nb, b8, 128)
    zs = t[:, 2].reshape(nb, b8, 128)

    nrm = jnp.stack([jnp.pad(aabb_min, (0, 1)),
                     jnp.pad(1.0 / (aabb_max - aabb_min), (0, 1))])
    mms = (mm_max - mm_min).reshape(1, 9)
    mmn = mm_min.reshape(1, 9)

    g1 = jnp.concatenate(
        [jnp.tile(W1[2 * l:2 * l + 2, :], (64, 1)) for l in range(_NUM_LEVELS)],
        axis=0)

    tabs = []
    for l in range(_NUM_LEVELS):
        s = _SIZES[l]
        flat = embeddings[l, :s].reshape(-1)
        pad = _ROWS[l] * 128 - 2 * s
        if pad:
            flat = jnp.concatenate([flat, jnp.zeros((pad,), flat.dtype)])
        tabs.append(flat.reshape(_ROWS[l], 1, 128))

    coord_spec = pl.BlockSpec((1, b8, 128), lambda i: (i, 0, 0))
    full = lambda shape: pl.BlockSpec(shape, lambda i: tuple(0 for _ in shape))

    out = pl.pallas_call(
        _encode_mlp_kernel,
        out_shape=jax.ShapeDtypeStruct((n, 9), jnp.float32),
        grid=(nb,),
        in_specs=[coord_spec, coord_spec, coord_spec,
                  pl.BlockSpec(memory_space=pltpu.SMEM),
                  full((1, 9)), full((1, 9)),
                  full((2048, 32)), full((32, 32)), full((32, 9))]
                 + [full((_ROWS[l], 1, 128)) for l in range(_NUM_LEVELS)],
        out_specs=pl.BlockSpec((b, 9), lambda i: (i, 0)),
        scratch_shapes=[
            pltpu.VMEM((_NUM_LEVELS, 8, b8, 128), jnp.int32),
            pltpu.VMEM((_NUM_LEVELS * b, 128), jnp.float32),
            pltpu.SMEM((_NUM_LEVELS, 8, b8, 128), jnp.int32),
            pltpu.SemaphoreType.DMA,
        ],
        compiler_params=pltpu.CompilerParams(
            dimension_semantics=("arbitrary",),
            vmem_limit_bytes=56 * 1024 * 1024,
        ),
        name="mlptexture3d",
    )(xs, ys, zs, nrm, mmn, mms, g1, W2, W3, *tabs)

    return out.reshape(*lead_shape, 9)


# speedup vs baseline: 1.0957x; 1.0957x over previous
import numpy as np
import jax
import jax.numpy as jnp
from jax.experimental import pallas as pl
from jax.experimental.pallas import tpu as pltpu

_NUM_LEVELS = 16
_BASE_RES = 16
_DESIRED_RES = 4096
_LOG2_T = 19
_T = 2 ** _LOG2_T
_SCALE = float(np.exp(np.log(_DESIRED_RES / _BASE_RES) / (_NUM_LEVELS - 1)))
_RES = [int(np.ceil(_BASE_RES * _SCALE ** l)) for l in range(_NUM_LEVELS)]
_SIZES = [min(r ** 3, _T) for r in _RES]
_ROWS = [(2 * s + 127) // 128 for s in _SIZES]
_P1 = np.uint32(2654435761)
_P2 = np.uint32(805459861)


def _encode_mlp_kernel(x_ref, y_ref, z_ref, nrm_ref, mmn_ref, mms_ref,
                       g1_ref, w2_ref, w3_ref, *rest):
    tabs = rest[:_NUM_LEVELS]
    out_ref = rest[_NUM_LEVELS]
    (pk_ref, ws_ref, acc_ref, pk_smem, ws_smem, sem1, sem2) = \
        rest[_NUM_LEVELS + 1:]

    b8 = x_ref.shape[1]
    npts = b8 * 128

    coords = []
    for d, ref in enumerate((x_ref, y_ref, z_ref)):
        c = ref[0]
        c = jnp.clip((c - nrm_ref[0, d]) * nrm_ref[1, d], 0.0, 1.0)
        coords.append(c)

    for l in range(_NUM_LEVELS):
        res = _RES[l]
        dense = res ** 3 <= _T
        pgs, frs = [], []
        for c in coords:
            pos = c * (res - 1)
            pg = jnp.floor(pos)
            frs.append(pos - pg)
            pgs.append(pg.astype(jnp.int32))
        omf = [1.0 - f for f in frs]
        for corner in range(8):
            offs = [(corner >> d) & 1 for d in range(3)]
            cc = [jnp.clip(pgs[d] + offs[d], 0, res - 1) for d in range(3)]
            if dense:
                e = cc[0] + res * (cc[1] + res * cc[2])
            else:
                h = (cc[0].astype(jnp.uint32)
                     ^ (cc[1].astype(jnp.uint32) * _P1)
                     ^ (cc[2].astype(jnp.uint32) * _P2))
                e = (h & np.uint32(_T - 1)).astype(jnp.int32)
            w = ((frs[0] if offs[0] else omf[0])
                 * (frs[1] if offs[1] else omf[1])
                 * (frs[2] if offs[2] else omf[2]))
            pk_ref[l, corner] = ((e & 63) << 15) | (e >> 6)
            ws_ref[l, corner] = w

    cp1 = pltpu.make_async_copy(pk_ref, pk_smem, sem1)
    cp2 = pltpu.make_async_copy(ws_ref, ws_smem, sem2)
    cp1.start()
    cp2.start()
    cp1.wait()
    cp2.wait()

    iota2 = jax.lax.broadcasted_iota(jnp.int32, (1, 128), 1) & ~1

    def point_body(p, _):
        s = p >> 7
        li = p & 127
        for l in range(_NUM_LEVELS):
            cs = []
            for corner in range(8):
                pk = pk_smem[l, corner, s, li]
                w = ws_smem[l, corner, s, li]
                row = tabs[l][pk & 16383]
                cs.append(jnp.where(iota2 == (pk >> 14), row, 0.0) * w)
            acc = ((cs[0] + cs[1]) + (cs[2] + cs[3])) \
                + ((cs[4] + cs[5]) + (cs[6] + cs[7]))
            acc_ref[pl.ds(l * npts + p, 1), :] = acc
        return _

    jax.lax.fori_loop(0, npts, point_body, None)

    h = jnp.zeros((npts, 32), jnp.float32)
    for l in range(_NUM_LEVELS):
        h = h + jnp.dot(acc_ref[l * npts:(l + 1) * npts, :],
                        g1_ref[l * 128:(l + 1) * 128, :],
                        preferred_element_type=jnp.float32)
    h = jnp.maximum(h, 0.0)
    h = jnp.maximum(jnp.dot(h, w2_ref[:], preferred_element_type=jnp.float32), 0.0)
    o = jnp.dot(h, w3_ref[:], preferred_element_type=jnp.float32)
    o = jax.nn.sigmoid(o)
    out_ref[:] = o * mms_ref[:] + mmn_ref[:]


def kernel(texc, aabb_min, aabb_max, embeddings, W1, W2, W3, mm_min, mm_max):
    lead_shape = texc.shape[:-1]
    t = texc.reshape(-1, 3)
    n = t.shape[0]
    b = 512 if n % 512 == 0 else 128
    b8 = b // 128
    nb = n // b

    xs = t[:, 0].reshape(nb, b8, 128)
    ys = t[:, 1].reshape(nb, b8, 128)
    zs = t[:, 2].reshape(nb, b8, 128)

    nrm = jnp.stack([jnp.pad(aabb_min, (0, 1)),
                     jnp.pad(1.0 / (aabb_max - aabb_min), (0, 1))])
    mms = (mm_max - mm_min).reshape(1, 9)
    mmn = mm_min.reshape(1, 9)

    g1 = jnp.concatenate(
        [jnp.tile(W1[2 * l:2 * l + 2, :], (64, 1)) for l in range(_NUM_LEVELS)],
        axis=0)

    tabs = []
    for l in range(_NUM_LEVELS):
        s = _SIZES[l]
        flat = embeddings[l, :s].reshape(-1)
        pad = _ROWS[l] * 128 - 2 * s
        if pad:
            flat = jnp.concatenate([flat, jnp.zeros((pad,), flat.dtype)])
        tabs.append(flat.reshape(_ROWS[l], 1, 128))

    coord_spec = pl.BlockSpec((1, b8, 128), lambda i: (i, 0, 0))
    full = lambda shape: pl.BlockSpec(shape, lambda i: tuple(0 for _ in shape))

    out = pl.pallas_call(
        _encode_mlp_kernel,
        out_shape=jax.ShapeDtypeStruct((n, 9), jnp.float32),
        grid=(nb,),
        in_specs=[coord_spec, coord_spec, coord_spec,
                  pl.BlockSpec(memory_space=pltpu.SMEM),
                  full((1, 9)), full((1, 9)),
                  full((2048, 32)), full((32, 32)), full((32, 9))]
                 + [full((_ROWS[l], 1, 128)) for l in range(_NUM_LEVELS)],
        out_specs=pl.BlockSpec((b, 9), lambda i: (i, 0)),
        scratch_shapes=[
            pltpu.VMEM((_NUM_LEVELS, 8, b8, 128), jnp.int32),
            pltpu.VMEM((_NUM_LEVELS, 8, b8, 128), jnp.float32),
            pltpu.VMEM((_NUM_LEVELS * b, 128), jnp.float32),
            pltpu.SMEM((_NUM_LEVELS, 8, b8, 128), jnp.int32),
            pltpu.SMEM((_NUM_LEVELS, 8, b8, 128), jnp.float32),
            pltpu.SemaphoreType.DMA,
            pltpu.SemaphoreType.DMA,
        ],
        compiler_params=pltpu.CompilerParams(
            dimension_semantics=("arbitrary",),
            vmem_limit_bytes=56 * 1024 * 1024,
        ),
        name="mlptexture3d",
    )(xs, ys, zs, nrm, mmn, mms, g1, W2, W3, *tabs)

    return out.reshape(*lead_shape, 9)
